# Initial kernel scaffold; baseline (speedup 1.0000x reference)
import jax
import jax.numpy as jnp
from jax.experimental import pallas as pl


def kernel(feature_matrix, edge_index, batch, W1, b1, W2, b2, W3, b3, Dw1, Db1, Dw2, Db2, Dw3, Db3):
    raise NotImplementedError("write your pallas kernel here")


# speedup vs baseline: 3.2338x; 3.2338x over previous
import functools

import jax
import jax.numpy as jnp
from jax import lax
from jax.experimental import pallas as pl
from jax.experimental.pallas import tpu as pltpu
from jax.experimental.pallas import tpu_sc as plsc

N_NODES = 10000
N_EDGES = 320000
F = 128
NUM_GRAPHS = 64

NS = 16
EPT = N_EDGES // NS
K = 80
NCHUNK = EPT // K
NPAIR = NCHUNK // 2
N_HALF = 5120
ACC_R = N_HALF + 8
WR = N_HALF // NS
LAST_WR = N_NODES - N_HALF - (NS - 1) * WR
ZROWS = 8

_MESH = plsc.VectorSubcoreMesh(core_axis_name="c", subcore_axis_name="s",
                               num_cores=1)


@functools.partial(
    pl.kernel,
    out_type=jax.ShapeDtypeStruct((N_NODES, F), jnp.float32),
    mesh=_MESH,
    scratch_types=[
        pltpu.VMEM((EPT,), jnp.int32),
        pltpu.VMEM((NCHUNK, K), jnp.int32),
        pltpu.VMEM((2, K, F), jnp.float32),
        pltpu.VMEM((ZROWS, F), jnp.float32),
        pltpu.VMEM_SHARED((ACC_R, F), jnp.float32),
        pltpu.SemaphoreType.DMA,
        pltpu.SemaphoreType.DMA,
    ],
)
def _spmm_sc(x_hbm, row_hbm, col_hbm, out_hbm,
             col_v, row_l, rows_v, zbuf, acc, sem0, sem1):
    s = lax.axis_index("s")
    base = s * WR

    def _zrow(i, carry):
        def _zcol(j, carry2):
            zbuf[i, pl.ds(j * 16, 16)] = jnp.zeros((16,), jnp.float32)
            return carry2
        return lax.fori_loop(0, F // 16, _zcol, carry, unroll=True)
    lax.fori_loop(0, ZROWS, _zrow, 0)

    pltpu.sync_copy(col_hbm.at[s], col_v)

    def _gather(g, buf, sem):
        return pltpu.async_copy(
            x_hbm.at[col_v.at[pl.ds(g * K, K)]], rows_v.at[buf], sem)

    def _pass(h, carry):
        lo = h * N_HALF

        def _zc(t, carry2):
            pltpu.sync_copy(zbuf, acc.at[pl.ds(base + t * ZROWS, ZROWS)])
            return carry2
        lax.fori_loop(0, WR // ZROWS, _zc, 0)

        pltpu.sync_copy(row_hbm.at[s], row_l)

        def _rm(q, carry2):
            def _rmj(j, carry3):
                v = row_l[q, pl.ds(j * 16, 16)]
                local = v - lo
                ok = (local >= 0) & (local < N_HALF)
                row_l[q, pl.ds(j * 16, 16)] = jnp.where(ok, local, N_HALF)
                return carry3
            return lax.fori_loop(0, K // 16, _rmj, carry2, unroll=True)
        lax.fori_loop(0, NCHUNK, _rm, 0)

        plsc.subcore_barrier()

        _gather(0, 0, sem0)

        def _pair(p, carry2):
            g0 = p * 2
            d1 = _gather(g0 + 1, 1, sem1)
            pltpu.make_async_copy(
                x_hbm.at[col_v.at[pl.ds(g0 * K, K)]], rows_v.at[0],
                sem0).wait()
            pltpu.sync_copy(rows_v.at[0], acc.at[row_l.at[g0]], add=True)

            @pl.when(p + 1 < NPAIR)
            def _():
                _gather(g0 + 2, 0, sem0)

            d1.wait()
            pltpu.sync_copy(rows_v.at[1], acc.at[row_l.at[g0 + 1]], add=True)
            return carry2

        lax.fori_loop(0, NPAIR, _pair, 0)

        plsc.subcore_barrier()

        @pl.when((s < NS - 1) | (h == 0))
        def _():
            pltpu.sync_copy(acc.at[pl.ds(base, WR)],
                            out_hbm.at[pl.ds(lo + base, WR)])

        @pl.when((s == NS - 1) & (h == 1))
        def _():
            pltpu.sync_copy(acc.at[pl.ds(base, LAST_WR)],
                            out_hbm.at[pl.ds(lo + base, LAST_WR)])
        return carry

    lax.fori_loop(0, 2, _pass, 0)


_BLK = 1000
_GRID = N_NODES // _BLK


def _mm_body(x_ref, b_ref, flag_ref, w_ref, o_ref):
    x = x_ref[...] + b_ref[...]
    x = jnp.where(flag_ref[0, 0] > 0, jnp.maximum(x, 0.0), x)
    o_ref[...] = jnp.dot(x, w_ref[...], preferred_element_type=jnp.float32)


_mm = pl.pallas_call(
    _mm_body,
    grid=(_GRID,),
    in_specs=[
        pl.BlockSpec((_BLK, F), lambda i: (i, 0)),
        pl.BlockSpec((1, F), lambda i: (0, 0)),
        pl.BlockSpec((1, 1), lambda i: (0, 0)),
        pl.BlockSpec((F, F), lambda i: (0, 0)),
    ],
    out_specs=pl.BlockSpec((_BLK, F), lambda i: (i, 0)),
    out_shape=jax.ShapeDtypeStruct((N_NODES, F), jnp.float32),
)


def _final_body(p_ref, b_ref, batch_ref, dw1, db1, dw2, db2, dw3, db3,
                o_ref, sums, counts):
    i = pl.program_id(0)

    @pl.when(i == 0)
    def _():
        sums[...] = jnp.zeros_like(sums)
        counts[...] = jnp.zeros_like(counts)

    x = p_ref[...] + b_ref[...]
    bb = batch_ref[0]
    ids = lax.broadcasted_iota(jnp.int32, (NUM_GRAPHS, _BLK), 0)
    oh = (ids == bb).astype(jnp.float32)
    sums[...] += jnp.dot(oh, x, preferred_element_type=jnp.float32)
    counts[...] += jnp.dot(oh, jnp.ones((_BLK, F), jnp.float32),
                           preferred_element_type=jnp.float32)

    @pl.when(i == pl.num_programs(0) - 1)
    def _():
        mean = sums[...] / jnp.maximum(counts[...], 1.0)
        z = jnp.maximum(
            jnp.dot(mean, dw1[...], preferred_element_type=jnp.float32)
            + db1[...], 0.0)
        z = jnp.maximum(
            jnp.dot(z, dw2[...], preferred_element_type=jnp.float32)
            + db2[...], 0.0)
        z = jnp.dot(z, dw3[...], preferred_element_type=jnp.float32) + db3[...]
        o_ref[...] = jax.nn.sigmoid(z)


_final = pl.pallas_call(
    _final_body,
    grid=(_GRID,),
    in_specs=[
        pl.BlockSpec((_BLK, F), lambda i: (i, 0)),
        pl.BlockSpec((1, F), lambda i: (0, 0)),
        pl.BlockSpec((1, 1, _BLK), lambda i: (i, 0, 0)),
        pl.BlockSpec((F, 16), lambda i: (0, 0)),
        pl.BlockSpec((1, 16), lambda i: (0, 0)),
        pl.BlockSpec((16, 8), lambda i: (0, 0)),
        pl.BlockSpec((1, 8), lambda i: (0, 0)),
        pl.BlockSpec((8, 1), lambda i: (0, 0)),
        pl.BlockSpec((1, 1), lambda i: (0, 0)),
    ],
    out_specs=pl.BlockSpec((NUM_GRAPHS, 1), lambda i: (0, 0)),
    out_shape=jax.ShapeDtypeStruct((NUM_GRAPHS, 1), jnp.float32),
    scratch_shapes=[
        pltpu.VMEM((NUM_GRAPHS, F), jnp.float32),
        pltpu.VMEM((NUM_GRAPHS, F), jnp.float32),
    ],
)


def kernel(feature_matrix, edge_index, batch, W1, b1, W2, b2, W3, b3,
           Dw1, Db1, Dw2, Db2, Dw3, Db3):
    ei = edge_index.astype(jnp.int32)
    row = ei[0].reshape(NS, NCHUNK, K)
    col = ei[1].reshape(NS, EPT)
    batch_r = batch.astype(jnp.int32).reshape(_GRID, 1, _BLK)

    w_stack = jnp.stack([W1, W2, W3])
    b_stack = jnp.stack([jnp.zeros_like(b1), b1, b2]).reshape(3, 1, F)
    flag_stack = jnp.array([0.0, 1.0, 1.0], jnp.float32).reshape(3, 1, 1)

    def _layer(y, xs):
        w, b, flag = xs
        h = _mm(y, b, flag, w)
        return _spmm_sc(h, row, col), None

    y, _ = lax.scan(_layer, feature_matrix, (w_stack, b_stack, flag_stack))
    return _final(y, b3.reshape(1, F), batch_r, Dw1, Db1.reshape(1, 16),
                  Dw2, Db2.reshape(1, 8), Dw3, Db3.reshape(1, 1))


# speedup vs baseline: 8.2035x; 2.5368x over previous
import functools

import jax
import jax.numpy as jnp
from jax import lax
from jax.experimental import pallas as pl
from jax.experimental.pallas import tpu as pltpu
from jax.experimental.pallas import tpu_sc as plsc

N_NODES = 10000
N_EDGES = 320000
F = 128
NUM_GRAPHS = 64

NS = 16
EPT = N_EDGES // NS
K = 40
NCHUNK = EPT // K
NBUF = 4
IBUF = 8
N_ACC = 10240
RPT = N_ACC // NS
LAST_WR = N_NODES - (NS - 1) * RPT
ZROWS = 8

_MESH = plsc.VectorSubcoreMesh(core_axis_name="c", subcore_axis_name="s",
                               num_cores=1)


@functools.partial(
    pl.kernel,
    out_type=jax.ShapeDtypeStruct((N_NODES, F), jnp.float32),
    mesh=_MESH,
    scratch_types=[
        pltpu.VMEM((IBUF, K), jnp.int32),
        pltpu.VMEM((IBUF, K), jnp.int32),
        pltpu.VMEM((NBUF, K, F), jnp.float32),
        pltpu.VMEM((ZROWS, F), jnp.float32),
        pltpu.VMEM_SHARED((N_ACC, F), jnp.float32),
        [pltpu.SemaphoreType.DMA] * IBUF,
        [pltpu.SemaphoreType.DMA] * IBUF,
        [pltpu.SemaphoreType.DMA] * NBUF,
    ],
)
def _spmm_sc(x_hbm, row_hbm, col_hbm, out_hbm, row_b, col_b, rows_v, zbuf,
             acc, sems_r, sems_c, sems_g):
    s = lax.axis_index("s")
    base = s * RPT

    def _zrow(i, carry):
        def _zcol(j, carry2):
            zbuf[i, pl.ds(j * 16, 16)] = jnp.zeros((16,), jnp.float32)
            return carry2
        return lax.fori_loop(0, F // 16, _zcol, carry, unroll=True)
    lax.fori_loop(0, ZROWS, _zrow, 0)

    def _zc(t, carry):
        pltpu.sync_copy(zbuf, acc.at[pl.ds(base + t * ZROWS, ZROWS)])
        return carry
    lax.fori_loop(0, RPT // ZROWS, _zc, 0)

    plsc.subcore_barrier()

    def _idx_load(g, slot):
        pltpu.async_copy(row_hbm.at[s, g], row_b.at[slot], sems_r[slot])
        pltpu.async_copy(col_hbm.at[s, g], col_b.at[slot], sems_c[slot])

    def _gather(islot, rslot):
        return pltpu.async_copy(x_hbm.at[col_b.at[islot]],
                                rows_v.at[rslot], sems_g[rslot])

    for slot in range(IBUF - NBUF + 1):
        _idx_load(slot, slot)

    D = NBUF - 1
    TOT = NCHUNK + D
    NBLK = (TOT + IBUF - 1) // IBUF

    def _blk(p, carry):
        for u in range(IBUF):
            g = p * IBUF + u
            iu = u % IBUF
            ru = u % NBUF

            @pl.when(g < NCHUNK)
            def _():
                pltpu.make_async_copy(col_hbm.at[s, g], col_b.at[iu],
                                      sems_c[iu]).wait()
                _gather(iu, ru)

            gd = g - D
            iud = (u - D) % IBUF
            rud = (u - D) % NBUF

            @pl.when((gd >= 0) & (gd < NCHUNK))
            def _():
                pltpu.make_async_copy(
                    x_hbm.at[col_b.at[iud]], rows_v.at[rud],
                    sems_g[rud]).wait()
                pltpu.make_async_copy(row_hbm.at[s, gd], row_b.at[iud],
                                      sems_r[iud]).wait()
                pltpu.sync_copy(rows_v.at[rud], acc.at[row_b.at[iud]],
                                add=True)

            gn = g + IBUF - D
            @pl.when(gn < NCHUNK)
            def _():
                _idx_load(gn, iud)
        return carry

    lax.fori_loop(0, NBLK, _blk, 0)

    plsc.subcore_barrier()

    @pl.when(s < NS - 1)
    def _():
        pltpu.sync_copy(acc.at[pl.ds(base, RPT)],
                        out_hbm.at[pl.ds(base, RPT)])

    @pl.when(s == NS - 1)
    def _():
        pltpu.sync_copy(acc.at[pl.ds(base, LAST_WR)],
                        out_hbm.at[pl.ds(base, LAST_WR)])


_BLK = 1000
_GRID = N_NODES // _BLK


def _mm_body(x_ref, b_ref, flag_ref, w_ref, o_ref):
    x = x_ref[...] + b_ref[...]
    x = jnp.where(flag_ref[0, 0] > 0, jnp.maximum(x, 0.0), x)
    o_ref[...] = jnp.dot(x, w_ref[...], preferred_element_type=jnp.float32)


_mm = pl.pallas_call(
    _mm_body,
    grid=(_GRID,),
    in_specs=[
        pl.BlockSpec((_BLK, F), lambda i: (i, 0)),
        pl.BlockSpec((1, F), lambda i: (0, 0)),
        pl.BlockSpec((1, 1), lambda i: (0, 0)),
        pl.BlockSpec((F, F), lambda i: (0, 0)),
    ],
    out_specs=pl.BlockSpec((_BLK, F), lambda i: (i, 0)),
    out_shape=jax.ShapeDtypeStruct((N_NODES, F), jnp.float32),
)


def _final_body(p_ref, b_ref, batch_ref, dw1, db1, dw2, db2, dw3, db3,
                o_ref, sums, counts):
    i = pl.program_id(0)

    @pl.when(i == 0)
    def _():
        sums[...] = jnp.zeros_like(sums)
        counts[...] = jnp.zeros_like(counts)

    x = p_ref[...] + b_ref[...]
    bb = batch_ref[0]
    ids = lax.broadcasted_iota(jnp.int32, (NUM_GRAPHS, _BLK), 0)
    oh = (ids == bb).astype(jnp.float32)
    sums[...] += jnp.dot(oh, x, preferred_element_type=jnp.float32)
    counts[...] += jnp.dot(oh, jnp.ones((_BLK, F), jnp.float32),
                           preferred_element_type=jnp.float32)

    @pl.when(i == pl.num_programs(0) - 1)
    def _():
        mean = sums[...] / jnp.maximum(counts[...], 1.0)
        z = jnp.maximum(
            jnp.dot(mean, dw1[...], preferred_element_type=jnp.float32)
            + db1[...], 0.0)
        z = jnp.maximum(
            jnp.dot(z, dw2[...], preferred_element_type=jnp.float32)
            + db2[...], 0.0)
        z = jnp.dot(z, dw3[...], preferred_element_type=jnp.float32) + db3[...]
        o_ref[...] = jax.nn.sigmoid(z)


_final = pl.pallas_call(
    _final_body,
    grid=(_GRID,),
    in_specs=[
        pl.BlockSpec((_BLK, F), lambda i: (i, 0)),
        pl.BlockSpec((1, F), lambda i: (0, 0)),
        pl.BlockSpec((1, 1, _BLK), lambda i: (i, 0, 0)),
        pl.BlockSpec((F, 16), lambda i: (0, 0)),
        pl.BlockSpec((1, 16), lambda i: (0, 0)),
        pl.BlockSpec((16, 8), lambda i: (0, 0)),
        pl.BlockSpec((1, 8), lambda i: (0, 0)),
        pl.BlockSpec((8, 1), lambda i: (0, 0)),
        pl.BlockSpec((1, 1), lambda i: (0, 0)),
    ],
    out_specs=pl.BlockSpec((NUM_GRAPHS, 1), lambda i: (0, 0)),
    out_shape=jax.ShapeDtypeStruct((NUM_GRAPHS, 1), jnp.float32),
    scratch_shapes=[
        pltpu.VMEM((NUM_GRAPHS, F), jnp.float32),
        pltpu.VMEM((NUM_GRAPHS, F), jnp.float32),
    ],
)


def kernel(feature_matrix, edge_index, batch, W1, b1, W2, b2, W3, b3,
           Dw1, Db1, Dw2, Db2, Dw3, Db3):
    ei = edge_index.astype(jnp.int32)
    row = ei[0].reshape(NS, NCHUNK, K)
    col = ei[1].reshape(NS, NCHUNK, K)
    batch_r = batch.astype(jnp.int32).reshape(_GRID, 1, _BLK)

    w_stack = jnp.stack([W1, W2, W3])
    b_stack = jnp.stack([jnp.zeros_like(b1), b1, b2]).reshape(3, 1, F)
    flag_stack = jnp.array([0.0, 1.0, 1.0], jnp.float32).reshape(3, 1, 1)

    def _layer(y, xs):
        w, b, flag = xs
        h = _mm(y, b, flag, w)
        return _spmm_sc(h, row, col), None

    y, _ = lax.scan(_layer, feature_matrix, (w_stack, b_stack, flag_stack))
    return _final(y, b3.reshape(1, F), batch_r, Dw1, Db1.reshape(1, 16),
                  Dw2, Db2.reshape(1, 8), Dw3, Db3.reshape(1, 1))


# speedup vs baseline: 8.4090x; 1.0250x over previous
import functools

import jax
import jax.numpy as jnp
from jax import lax
from jax.experimental import pallas as pl
from jax.experimental.pallas import tpu as pltpu
from jax.experimental.pallas import tpu_sc as plsc

N_NODES = 10000
N_EDGES = 320000
F = 128
NUM_GRAPHS = 64

NS = 16
EPT = N_EDGES // NS
K = 40
NCHUNK = EPT // K
NBUF = 4
IBUF = 8
N_ACC = 10240
RPT = N_ACC // NS
LAST_WR = N_NODES - (NS - 1) * RPT
ZROWS = 16

_MESH = plsc.VectorSubcoreMesh(core_axis_name="c", subcore_axis_name="s",
                               num_cores=1)


@functools.partial(
    pl.kernel,
    out_type=jax.ShapeDtypeStruct((N_NODES, F), jnp.float32),
    mesh=_MESH,
    scratch_types=[
        pltpu.VMEM((IBUF, K), jnp.int32),
        pltpu.VMEM((IBUF, K), jnp.int32),
        pltpu.VMEM((NBUF, K, F), jnp.float32),
        pltpu.VMEM((ZROWS, F), jnp.float32),
        pltpu.VMEM_SHARED((N_ACC, F), jnp.float32),
        [pltpu.SemaphoreType.DMA] * IBUF,
        [pltpu.SemaphoreType.DMA] * IBUF,
        [pltpu.SemaphoreType.DMA] * NBUF,
        pltpu.SemaphoreType.DMA,
    ],
)
def _spmm_sc(x_hbm, row_hbm, col_hbm, out_hbm, row_b, col_b, rows_v, zbuf,
             acc, sems_r, sems_c, sems_g, zsem):
    s = lax.axis_index("s")
    base = s * RPT

    def _zrow(i, carry):
        def _zcol(j, carry2):
            zbuf[i, pl.ds(j * 16, 16)] = jnp.zeros((16,), jnp.float32)
            return carry2
        return lax.fori_loop(0, F // 16, _zcol, carry, unroll=True)
    lax.fori_loop(0, ZROWS, _zrow, 0)

    def _idx_load(g, slot):
        pltpu.async_copy(row_hbm.at[s, g], row_b.at[slot], sems_r[slot])
        pltpu.async_copy(col_hbm.at[s, g], col_b.at[slot], sems_c[slot])

    def _gather(islot, rslot):
        return pltpu.async_copy(x_hbm.at[col_b.at[islot]],
                                rows_v.at[rslot], sems_g[rslot])

    NZ = RPT // ZROWS
    for t in range(NZ):
        pltpu.async_copy(zbuf, acc.at[pl.ds(base + t * ZROWS, ZROWS)], zsem)

    for slot in range(IBUF - NBUF + 1):
        _idx_load(slot, slot)

    for t in range(NZ):
        pltpu.make_async_copy(zbuf, acc.at[pl.ds(base + t * ZROWS, ZROWS)],
                              zsem).wait()

    plsc.subcore_barrier()


    D = NBUF - 1
    TOT = NCHUNK + D
    NBLK = (TOT + IBUF - 1) // IBUF

    def _blk(p, carry):
        for u in range(IBUF):
            g = p * IBUF + u
            iu = u % IBUF
            ru = u % NBUF

            @pl.when(g < NCHUNK)
            def _():
                pltpu.make_async_copy(col_hbm.at[s, g], col_b.at[iu],
                                      sems_c[iu]).wait()
                _gather(iu, ru)

            gd = g - D
            iud = (u - D) % IBUF
            rud = (u - D) % NBUF

            @pl.when((gd >= 0) & (gd < NCHUNK))
            def _():
                pltpu.make_async_copy(
                    x_hbm.at[col_b.at[iud]], rows_v.at[rud],
                    sems_g[rud]).wait()
                pltpu.make_async_copy(row_hbm.at[s, gd], row_b.at[iud],
                                      sems_r[iud]).wait()
                pltpu.sync_copy(rows_v.at[rud], acc.at[row_b.at[iud]],
                                add=True)

            gn = g + IBUF - D
            @pl.when(gn < NCHUNK)
            def _():
                _idx_load(gn, iud)
        return carry

    lax.fori_loop(0, NBLK, _blk, 0)

    plsc.subcore_barrier()

    @pl.when(s < NS - 1)
    def _():
        pltpu.sync_copy(acc.at[pl.ds(base, RPT)],
                        out_hbm.at[pl.ds(base, RPT)])

    @pl.when(s == NS - 1)
    def _():
        pltpu.sync_copy(acc.at[pl.ds(base, LAST_WR)],
                        out_hbm.at[pl.ds(base, LAST_WR)])


_BLK = 1000
_GRID = N_NODES // _BLK


def _mm_body(x_ref, b_ref, flag_ref, w_ref, o_ref):
    x = x_ref[...] + b_ref[...]
    x = jnp.where(flag_ref[0, 0] > 0, jnp.maximum(x, 0.0), x)
    o_ref[...] = jnp.dot(x, w_ref[...], preferred_element_type=jnp.float32)


_mm = pl.pallas_call(
    _mm_body,
    grid=(_GRID,),
    in_specs=[
        pl.BlockSpec((_BLK, F), lambda i: (i, 0)),
        pl.BlockSpec((1, F), lambda i: (0, 0)),
        pl.BlockSpec((1, 1), lambda i: (0, 0)),
        pl.BlockSpec((F, F), lambda i: (0, 0)),
    ],
    out_specs=pl.BlockSpec((_BLK, F), lambda i: (i, 0)),
    out_shape=jax.ShapeDtypeStruct((N_NODES, F), jnp.float32),
)


def _final_body(p_ref, b_ref, batch_ref, dw1, db1, dw2, db2, dw3, db3,
                o_ref, sums, counts):
    i = pl.program_id(0)

    @pl.when(i == 0)
    def _():
        sums[...] = jnp.zeros_like(sums)
        counts[...] = jnp.zeros_like(counts)

    x = p_ref[...] + b_ref[...]
    bb = batch_ref[0]
    ids = lax.broadcasted_iota(jnp.int32, (NUM_GRAPHS, _BLK), 0)
    oh = (ids == bb).astype(jnp.float32)
    sums[...] += jnp.dot(oh, x, preferred_element_type=jnp.float32)
    counts[...] += jnp.dot(oh, jnp.ones((_BLK, F), jnp.float32),
                           preferred_element_type=jnp.float32)

    @pl.when(i == pl.num_programs(0) - 1)
    def _():
        mean = sums[...] / jnp.maximum(counts[...], 1.0)
        z = jnp.maximum(
            jnp.dot(mean, dw1[...], preferred_element_type=jnp.float32)
            + db1[...], 0.0)
        z = jnp.maximum(
            jnp.dot(z, dw2[...], preferred_element_type=jnp.float32)
            + db2[...], 0.0)
        z = jnp.dot(z, dw3[...], preferred_element_type=jnp.float32) + db3[...]
        o_ref[...] = jax.nn.sigmoid(z)


_final = pl.pallas_call(
    _final_body,
    grid=(_GRID,),
    in_specs=[
        pl.BlockSpec((_BLK, F), lambda i: (i, 0)),
        pl.BlockSpec((1, F), lambda i: (0, 0)),
        pl.BlockSpec((1, 1, _BLK), lambda i: (i, 0, 0)),
        pl.BlockSpec((F, 16), lambda i: (0, 0)),
        pl.BlockSpec((1, 16), lambda i: (0, 0)),
        pl.BlockSpec((16, 8), lambda i: (0, 0)),
        pl.BlockSpec((1, 8), lambda i: (0, 0)),
        pl.BlockSpec((8, 1), lambda i: (0, 0)),
        pl.BlockSpec((1, 1), lambda i: (0, 0)),
    ],
    out_specs=pl.BlockSpec((NUM_GRAPHS, 1), lambda i: (0, 0)),
    out_shape=jax.ShapeDtypeStruct((NUM_GRAPHS, 1), jnp.float32),
    scratch_shapes=[
        pltpu.VMEM((NUM_GRAPHS, F), jnp.float32),
        pltpu.VMEM((NUM_GRAPHS, F), jnp.float32),
    ],
)


def kernel(feature_matrix, edge_index, batch, W1, b1, W2, b2, W3, b3,
           Dw1, Db1, Dw2, Db2, Dw3, Db3):
    ei = edge_index.astype(jnp.int32)
    row = ei[0].reshape(NS, NCHUNK, K)
    col = ei[1].reshape(NS, NCHUNK, K)
    batch_r = batch.astype(jnp.int32).reshape(_GRID, 1, _BLK)

    w_stack = jnp.stack([W1, W2, W3])
    b_stack = jnp.stack([jnp.zeros_like(b1), b1, b2]).reshape(3, 1, F)
    flag_stack = jnp.array([0.0, 1.0, 1.0], jnp.float32).reshape(3, 1, 1)

    def _layer(y, xs):
        w, b, flag = xs
        h = _mm(y, b, flag, w)
        return _spmm_sc(h, row, col), None

    y, _ = lax.scan(_layer, feature_matrix, (w_stack, b_stack, flag_stack))
    return _final(y, b3.reshape(1, F), batch_r, Dw1, Db1.reshape(1, 16),
                  Dw2, Db2.reshape(1, 8), Dw3, Db3.reshape(1, 1))
